# Initial kernel scaffold; baseline (speedup 1.0000x reference)
import jax
import jax.numpy as jnp
from jax.experimental import pallas as pl


def kernel(x, indices):
    raise NotImplementedError("write your pallas kernel here")


# speedup vs baseline: 1.9586x; 1.9586x over previous
import jax
import jax.numpy as jnp
from jax.experimental import pallas as pl
from jax.experimental.pallas import tpu as pltpu


def _body(idx_ref, x_ref, o_ref):
    c = x_ref.shape[1]
    k = o_ref.shape[1]
    col = jax.lax.broadcasted_iota(jnp.int32, (c, k), 0)
    onehot = (col == idx_ref[0, :][None, :]).astype(jnp.float32)
    o_ref[...] = jnp.dot(x_ref[...], onehot, preferred_element_type=jnp.float32)


def kernel(x, indices):
    b, s, c = x.shape
    k = indices.shape[0]
    rows = b * s
    x2 = x.reshape(rows, c)
    blk = 256
    out = pl.pallas_call(
        _body,
        grid=(rows // blk,),
        in_specs=[
            pl.BlockSpec((1, k), lambda i: (0, 0)),
            pl.BlockSpec((blk, c), lambda i: (i, 0)),
        ],
        out_specs=pl.BlockSpec((blk, k), lambda i: (i, 0)),
        out_shape=jax.ShapeDtypeStruct((rows, k), jnp.float32),
    )(indices.reshape(1, k), x2)
    return out.reshape(b, s, k)
